# Initial kernel scaffold; baseline (speedup 1.0000x reference)
import jax
import jax.numpy as jnp
from jax.experimental import pallas as pl


def kernel(x, Wg, bg, W1, b1, W2, b2):
    raise NotImplementedError("write your pallas kernel here")


# speedup vs baseline: 2.8449x; 2.8449x over previous
import jax
import jax.numpy as jnp
from jax.experimental import pallas as pl
from jax.experimental.pallas import tpu as pltpu

_TB = 2048


def _moe_kernel(x_ref, wg_ref, bg_ref, w1_ref, b1_ref, w2_ref, b2_ref, out_ref):
    e = pl.program_id(1)
    n_e = pl.num_programs(1)
    x = x_ref[...]
    glog = jnp.dot(x, wg_ref[...], preferred_element_type=jnp.float32) + bg_ref[...]
    idx = jax.lax.broadcasted_iota(jnp.int32, glog.shape, 1)
    m1 = jnp.max(glog, axis=1, keepdims=True)
    i1 = jnp.min(jnp.where(glog >= m1, idx, n_e), axis=1, keepdims=True)
    neg = jnp.finfo(jnp.float32).min
    g2 = jnp.where(idx == i1, neg, glog)
    m2 = jnp.max(g2, axis=1, keepdims=True)
    i2 = jnp.min(jnp.where(g2 >= m2, idx, n_e), axis=1, keepdims=True)
    p2 = jnp.exp(m2 - m1)
    denom = 1.0 + p2
    we = jnp.where(i1 == e, 1.0 / denom, jnp.where(i2 == e, p2 / denom, 0.0))
    h = jnp.maximum(
        jnp.dot(x, w1_ref[0], preferred_element_type=jnp.float32) + b1_ref[0], 0.0)
    y = jnp.dot(h, w2_ref[0], preferred_element_type=jnp.float32) + b2_ref[0]
    contrib = we * y

    @pl.when(e == 0)
    def _init():
        out_ref[...] = contrib

    @pl.when(e != 0)
    def _acc():
        out_ref[...] += contrib


def kernel(x, Wg, bg, W1, b1, W2, b2):
    B, D = x.shape
    E = Wg.shape[1]
    nb = B // _TB
    out = pl.pallas_call(
        _moe_kernel,
        grid=(nb, E),
        in_specs=[
            pl.BlockSpec((_TB, D), lambda i, e: (i, 0)),
            pl.BlockSpec((D, E), lambda i, e: (0, 0)),
            pl.BlockSpec((1, E), lambda i, e: (0, 0)),
            pl.BlockSpec((1, D, D), lambda i, e: (e, 0, 0)),
            pl.BlockSpec((1, 1, D), lambda i, e: (e, 0, 0)),
            pl.BlockSpec((1, D, D), lambda i, e: (e, 0, 0)),
            pl.BlockSpec((1, 1, D), lambda i, e: (e, 0, 0)),
        ],
        out_specs=pl.BlockSpec((_TB, D), lambda i, e: (i, 0)),
        out_shape=jax.ShapeDtypeStruct((B, D), jnp.float32),
        compiler_params=pltpu.CompilerParams(
            dimension_semantics=("parallel", "arbitrary")),
    )(x, Wg, bg.reshape(1, E), W1, b1.reshape(E, 1, D), W2, b2.reshape(E, 1, D))
    return out
